# baseline (device time: 23825 ns/iter reference)
import jax
import jax.numpy as jnp
from jax import lax
from jax.experimental import pallas as pl
from jax.experimental.pallas import tpu as pltpu

N_DEV = 16
EPS = 1e-5
NB = 4


def kernel(x, gamma):
    m, n_per = x.shape
    mb = m // NB
    n_total = n_per * N_DEV
    g2 = gamma.reshape(1, n_per)

    def body(
        x_hbm,
        g_ref,
        out_hbm,
        xv,
        ov,
        comm_ref,
        in_sems,
        out_sems,
        send_sems,
        recv_sems,
    ):
        my = lax.axis_index("i")
        g = g_ref[0, :]

        in_copies = []
        for b in range(NB):
            c = pltpu.make_async_copy(
                x_hbm.at[pl.ds(b * mb, mb), :],
                xv.at[pl.ds(b * mb, mb), :],
                in_sems.at[b],
            )
            c.start()
            in_copies.append(c)

        rdmas = []
        for b in range(NB):
            in_copies[b].wait()
            seg = pl.ds(b * mb, mb)
            xb = xv[seg, :]
            comm_ref[0, seg] = jnp.sum(xb * xb, axis=1)
            for d in range(1, N_DEV):
                rdma = pltpu.make_async_remote_copy(
                    src_ref=comm_ref.at[0, seg],
                    dst_ref=comm_ref.at[d, seg],
                    send_sem=send_sems.at[b, d - 1],
                    recv_sem=recv_sems.at[b, d - 1],
                    device_id=((my + d) % N_DEV,),
                    device_id_type=pl.DeviceIdType.MESH,
                )
                rdma.start()
                rdmas.append(rdma)

        out_copies = []
        for b in range(NB):
            seg = pl.ds(b * mb, mb)
            for d in range(1, N_DEV):
                rdmas[b * (N_DEV - 1) + (d - 1)].wait_recv()
            tot = jnp.sum(comm_ref[:, seg], axis=0)
            inv = lax.rsqrt(tot * (1.0 / n_total) + EPS)
            ov[seg, :] = ((xv[seg, :] * inv[:, None]) * g[None, :]).astype(
                ov.dtype
            )
            oc = pltpu.make_async_copy(
                ov.at[seg, :], out_hbm.at[seg, :], out_sems.at[b]
            )
            oc.start()
            out_copies.append(oc)

        for oc in out_copies:
            oc.wait()
        for rdma in rdmas:
            rdma.wait_send()

    return pl.pallas_call(
        body,
        out_shape=jax.ShapeDtypeStruct((m, n_per), jnp.bfloat16),
        in_specs=[
            pl.BlockSpec(memory_space=pl.ANY),
            pl.BlockSpec(memory_space=pltpu.VMEM),
        ],
        out_specs=pl.BlockSpec(memory_space=pl.ANY),
        scratch_shapes=[
            pltpu.VMEM((m, n_per), jnp.float32),
            pltpu.VMEM((m, n_per), jnp.bfloat16),
            pltpu.VMEM((N_DEV, m), jnp.float32),
            pltpu.SemaphoreType.DMA((NB,)),
            pltpu.SemaphoreType.DMA((NB,)),
            pltpu.SemaphoreType.DMA((NB, N_DEV - 1)),
            pltpu.SemaphoreType.DMA((NB, N_DEV - 1)),
        ],
    )(x, g2)


# device time: 15905 ns/iter; 1.4980x vs baseline; 1.4980x over previous
import jax
import jax.numpy as jnp
from jax import lax
from jax.experimental import pallas as pl
from jax.experimental.pallas import tpu as pltpu

N_DEV = 16
EPS = 1e-5


def kernel(x, gamma):
    m, n_per = x.shape
    n_total = n_per * N_DEV
    g2 = gamma.reshape(1, n_per)

    def body(x_ref, g_ref, out_ref, comm_ref, send_sems, recv_sems):
        my = lax.axis_index("i")

        barrier_sem = pltpu.get_barrier_semaphore()
        for d in range(1, N_DEV):
            pl.semaphore_signal(
                barrier_sem,
                inc=1,
                device_id=((my + d) % N_DEV,),
                device_id_type=pl.DeviceIdType.MESH,
            )

        xx = x_ref[:, :]
        part = jnp.sum(xx * xx, axis=1)
        comm_ref[0, :] = part

        pl.semaphore_wait(barrier_sem, N_DEV - 1)

        rdmas = []
        for d in range(1, N_DEV):
            rdma = pltpu.make_async_remote_copy(
                src_ref=comm_ref.at[0],
                dst_ref=comm_ref.at[d],
                send_sem=send_sems.at[d - 1],
                recv_sem=recv_sems.at[d - 1],
                device_id=((my + d) % N_DEV,),
                device_id_type=pl.DeviceIdType.MESH,
            )
            rdma.start()
            rdmas.append(rdma)

        xg = xx * g_ref[0, :][None, :]

        for rdma in rdmas:
            rdma.wait_recv()
        tot = jnp.sum(comm_ref[:, :], axis=0)
        inv = lax.rsqrt(tot * (1.0 / n_total) + EPS)
        out_ref[:, :] = (xg * inv[:, None]).astype(out_ref.dtype)

        for rdma in rdmas:
            rdma.wait_send()

    return pl.pallas_call(
        body,
        out_shape=jax.ShapeDtypeStruct((m, n_per), jnp.bfloat16),
        in_specs=[
            pl.BlockSpec(memory_space=pltpu.VMEM),
            pl.BlockSpec(memory_space=pltpu.VMEM),
        ],
        out_specs=pl.BlockSpec(memory_space=pltpu.VMEM),
        scratch_shapes=[
            pltpu.VMEM((N_DEV, m), jnp.float32),
            pltpu.SemaphoreType.DMA((N_DEV - 1,)),
            pltpu.SemaphoreType.DMA((N_DEV - 1,)),
        ],
        compiler_params=pltpu.CompilerParams(collective_id=0),
    )(x, g2)
